# baseline (speedup 1.0000x reference)
import jax
import jax.numpy as jnp
from jax.experimental import pallas as pl
from jax.experimental.pallas import tpu as pltpu

_K = 20


def _graph_kernel(x_ref, xr_ref, phys_ref, w_ref, b_ref, alpha_ref, out_ref,
                  base_ref):
    RB = out_ref.shape[1]
    N = out_ref.shape[2]

    state = jnp.mean(x_ref[0], axis=0)
    state_r = jnp.mean(xr_ref[0], axis=0)
    wv = w_ref[0]
    bv = b_ref[0]
    emb = jnp.tanh(state[:, None] * wv[None, :] + bv[None, :])
    rows = jnp.tanh(state_r[:, None] * wv[None, :] + bv[None, :])

    alpha_c = jax.nn.sigmoid(alpha_ref[0, 0])
    one_minus = 1.0 - alpha_c

    @pl.when(pl.program_id(1) == 0)
    def _():
        phys = phys_ref[...]
        base_ref[...] = alpha_c * (
            phys / (jnp.sum(phys, axis=1, keepdims=True) + 1e-8))

    base = base_ref[...]

    a = jax.lax.dot_general(
        rows, emb, (((1,), (1,)), ((), ())),
        preferred_element_type=jnp.float32)
    a = jnp.maximum(a, 0.0)

    masked = a
    rowmax = jnp.max(a, axis=1, keepdims=True)
    m = rowmax
    vals = []
    cnts = []
    for i in range(_K):
        e = masked == m
        vals.append(m)
        cnts.append(jnp.sum(jnp.where(e, 1.0, 0.0), axis=1, keepdims=True))
        if i < _K - 1:
            masked = jnp.where(e, -jnp.inf, masked)
            m = jnp.max(masked, axis=1, keepdims=True)

    d = jnp.concatenate(vals, axis=1)
    c = jnp.concatenate(cnts, axis=1)
    cum = c
    sh = 1
    while sh < _K:
        cum = cum + jnp.concatenate(
            [jnp.zeros((RB, sh), jnp.float32), cum[:, :_K - sh]], axis=1)
        sh *= 2
    excl = cum - c
    kf = float(_K)
    hit = (cum >= kf) & (excl < kf)
    t = jnp.sum(jnp.where(hit, d, 0.0), axis=1, keepdims=True)
    r = kf - jnp.sum(jnp.where(hit, excl, 0.0), axis=1, keepdims=True)

    S = N // 128
    et = a == t
    et3 = jnp.where(et, 1.0, 0.0).reshape(RB * S, 128)
    lt = jnp.where(
        jax.lax.broadcasted_iota(jnp.int32, (128, 128), 0)
        <= jax.lax.broadcasted_iota(jnp.int32, (128, 128), 1),
        1.0, 0.0)
    rank3 = jax.lax.dot_general(
        et3, lt, (((1,), (0,)), ((), ())),
        preferred_element_type=jnp.float32)
    seg_tot = rank3[:, 127:128].reshape(RB, S)
    cum_seg = seg_tot
    sh = 1
    while sh < S:
        cum_seg = cum_seg + jnp.concatenate(
            [jnp.zeros((RB, sh), jnp.float32), cum_seg[:, :S - sh]], axis=1)
        sh *= 2
    off = (cum_seg - seg_tot).reshape(RB * S, 1)
    rank = (rank3 + off).reshape(RB, N)

    sel = (a > t) | (et & (rank <= r))
    p = jnp.where(sel, jnp.exp(a - rowmax), 0.0)
    z = jnp.sum(p, axis=1, keepdims=True)
    p = p / z

    out_ref[0] = base + one_minus * p


def kernel(x, A_physical, W, b, alpha):
    B, T, N, _ = x.shape
    RB = 512
    x3 = x[..., 0]
    wv = W[:, 0].reshape(1, 16)
    bv = b.reshape(1, 16)
    al = alpha.reshape(1, 1)

    return pl.pallas_call(
        _graph_kernel,
        grid=(N // RB, B),
        in_specs=[
            pl.BlockSpec((1, T, N), lambda r, b: (b, 0, 0)),
            pl.BlockSpec((1, T, RB), lambda r, b: (b, 0, r)),
            pl.BlockSpec((RB, N), lambda r, b: (r, 0)),
            pl.BlockSpec((1, 16), lambda r, b: (0, 0)),
            pl.BlockSpec((1, 16), lambda r, b: (0, 0)),
            pl.BlockSpec((1, 1), lambda r, b: (0, 0)),
        ],
        out_specs=pl.BlockSpec((1, RB, N), lambda r, b: (b, r, 0)),
        out_shape=jax.ShapeDtypeStruct((B, N, N), jnp.float32),
        scratch_shapes=[pltpu.VMEM((RB, N), jnp.float32)],
    )(x3, x3, A_physical, wv, bv, al)


# speedup vs baseline: 1.1324x; 1.1324x over previous
import jax
import jax.numpy as jnp
from jax.experimental import pallas as pl
from jax.experimental.pallas import tpu as pltpu

_K = 20


def _graph_kernel(x_ref, xr_ref, phys_ref, w_ref, b_ref, alpha_ref, out_ref,
                  base_ref):
    RB = out_ref.shape[1]
    N = out_ref.shape[2]

    state = jnp.mean(x_ref[0], axis=0)
    state_r = jnp.mean(xr_ref[0], axis=0)
    wv = w_ref[0]
    bv = b_ref[0]
    emb = jnp.tanh(state[:, None] * wv[None, :] + bv[None, :])
    rows = jnp.tanh(state_r[:, None] * wv[None, :] + bv[None, :])

    alpha_c = jax.nn.sigmoid(alpha_ref[0, 0])
    one_minus = 1.0 - alpha_c

    @pl.when(pl.program_id(1) == 0)
    def _():
        phys = phys_ref[...]
        base_ref[...] = alpha_c * (
            phys / (jnp.sum(phys, axis=1, keepdims=True) + 1e-8))

    base = base_ref[...]

    a = jax.lax.dot_general(
        rows, emb, (((1,), (1,)), ((), ())),
        preferred_element_type=jnp.float32)
    a = jnp.maximum(a, 0.0)

    masked = a
    rowmax = jnp.max(a, axis=1, keepdims=True)
    m = rowmax
    vals = []
    cnts = []
    for i in range(_K):
        e = masked == m
        vals.append(m)
        cnts.append(jnp.sum(jnp.where(e, 1.0, 0.0), axis=1, keepdims=True))
        if i < _K - 1:
            masked = jnp.where(e, -jnp.inf, masked)
            m = jnp.max(masked, axis=1, keepdims=True)

    d = jnp.concatenate(vals, axis=1)
    c = jnp.concatenate(cnts, axis=1)
    cum = c
    sh = 1
    while sh < _K:
        cum = cum + jnp.concatenate(
            [jnp.zeros((RB, sh), jnp.float32), cum[:, :_K - sh]], axis=1)
        sh *= 2
    excl = cum - c
    kf = float(_K)
    hit = (cum >= kf) & (excl < kf)
    t = jnp.sum(jnp.where(hit, d, 0.0), axis=1, keepdims=True)
    r = kf - jnp.sum(jnp.where(hit, excl, 0.0), axis=1, keepdims=True)

    et = a == t
    rank = jnp.where(et, 1.0, 0.0)
    sh = 1
    while sh < N:
        rank = rank + jnp.concatenate(
            [jnp.zeros((RB, sh), jnp.float32), rank[:, :N - sh]], axis=1)
        sh *= 2

    sel = (a > t) | (et & (rank <= r))
    p = jnp.where(sel, jnp.exp(a - rowmax), 0.0)
    z = jnp.sum(p, axis=1, keepdims=True)
    p = p / z

    out_ref[0] = base + one_minus * p


def kernel(x, A_physical, W, b, alpha):
    B, T, N, _ = x.shape
    RB = 512
    x3 = x[..., 0]
    wv = W[:, 0].reshape(1, 16)
    bv = b.reshape(1, 16)
    al = alpha.reshape(1, 1)

    return pl.pallas_call(
        _graph_kernel,
        grid=(N // RB, B),
        in_specs=[
            pl.BlockSpec((1, T, N), lambda r, b: (b, 0, 0)),
            pl.BlockSpec((1, T, RB), lambda r, b: (b, 0, r)),
            pl.BlockSpec((RB, N), lambda r, b: (r, 0)),
            pl.BlockSpec((1, 16), lambda r, b: (0, 0)),
            pl.BlockSpec((1, 16), lambda r, b: (0, 0)),
            pl.BlockSpec((1, 1), lambda r, b: (0, 0)),
        ],
        out_specs=pl.BlockSpec((1, RB, N), lambda r, b: (b, r, 0)),
        out_shape=jax.ShapeDtypeStruct((B, N, N), jnp.float32),
        scratch_shapes=[pltpu.VMEM((RB, N), jnp.float32)],
    )(x3, x3, A_physical, wv, bv, al)
